# baseline (speedup 1.0000x reference)
import jax
import jax.numpy as jnp
from jax import lax
from jax.experimental import pallas as pl
from jax.experimental.pallas import tpu as pltpu

_NUM_NEG = 4
_EPS = 1e-8
_BLK = 512
_HALF = 256
_NEG_INF = -1e30


def _bce0(v):
    return jnp.maximum(v, 0.0) + jnp.log1p(jnp.exp(-jnp.abs(v)))


def _loss_kernel(a_ref, p_full_ref, p_blk_ref, out_ref, pnn_ref, work_ref):
    i = pl.program_id(0)
    nsteps = pl.num_programs(0)
    blk, half, b = _BLK, _HALF, p_full_ref.shape[1]

    @pl.when(i == 0)
    def _prep():
        pt = p_full_ref[...]
        p_norm = jnp.sqrt(jnp.sum(pt * pt, axis=0, keepdims=True))
        p_nt = pt / p_norm
        pn_norm = jnp.sqrt(jnp.sum(p_nt * p_nt, axis=0, keepdims=True))
        pnn_ref[...] = (p_nt / pn_norm).astype(jnp.bfloat16)

    eye = (lax.broadcasted_iota(jnp.int32, (half, half), 0)
           == lax.broadcasted_iota(jnp.int32, (half, half), 1))

    def _half_terms(h):
        at = a_ref[:, h * half:(h + 1) * half]
        a_norm = jnp.sqrt(jnp.sum(at * at, axis=0, keepdims=True))
        a_nt = (at / a_norm).astype(jnp.bfloat16)

        scaled = lax.dot_general(pnn_ref[...], a_nt,
                                 (((0,), (0,)), ((), ())),
                                 preferred_element_type=jnp.float32
                                 ).astype(jnp.bfloat16)

        wslice = work_ref.at[:, h * half:(h + 1) * half]
        wslice[...] = scaled
        base = i * blk + h * half
        sub = wslice[pl.ds(base, half), :]
        wslice[pl.ds(base, half), :] = jnp.where(
            eye, jnp.bfloat16(_NEG_INF), sub)
        work = wslice[...]

        v1 = jnp.max(work, axis=0, keepdims=True)
        v2 = jnp.max(jnp.where(work < v1, work, jnp.bfloat16(_NEG_INF)),
                     axis=0, keepdims=True)
        v3 = jnp.max(jnp.where(work < v2, work, jnp.bfloat16(_NEG_INF)),
                     axis=0, keepdims=True)
        v4 = jnp.max(jnp.where(work < v3, work, jnp.bfloat16(_NEG_INF)),
                     axis=0, keepdims=True)
        v1, v2, v3, v4 = (v.astype(jnp.float32) for v in (v1, v2, v3, v4))

        pbt = p_blk_ref[:, h * half:(h + 1) * half]
        num = jnp.sum(at * pbt, axis=0, keepdims=True)
        pb_norm = jnp.sqrt(jnp.sum(pbt * pbt, axis=0, keepdims=True))
        x = num / jnp.clip(a_norm * pb_norm, _EPS, None)
        pos = jnp.maximum(x, 0.0) - x + jnp.log1p(jnp.exp(-jnp.abs(x)))

        t4 = jnp.where(v4 < 0.0, _bce0(x), _bce0(v4))
        return jnp.sum(pos + _bce0(v1) + _bce0(v2) + _bce0(v3) + t4)

    total = _half_terms(0) + _half_terms(1)

    @pl.when(i == 0)
    def _init():
        out_ref[...] = jnp.zeros((1, 1), jnp.float32)

    out_ref[...] += jnp.full((1, 1), total, jnp.float32)

    @pl.when(i == nsteps - 1)
    def _finish():
        out_ref[...] = out_ref[...] / (b * (_NUM_NEG + 1))


def kernel(anchor, positive):
    b, d = positive.shape
    at = anchor.T
    pt = positive.T
    grid = b // _BLK
    out = pl.pallas_call(
        _loss_kernel,
        grid=(grid,),
        in_specs=[
            pl.BlockSpec((d, _BLK), lambda i: (0, i)),
            pl.BlockSpec((d, b), lambda i: (0, 0)),
            pl.BlockSpec((d, _BLK), lambda i: (0, i)),
        ],
        out_specs=pl.BlockSpec((1, 1), lambda i: (0, 0)),
        out_shape=jax.ShapeDtypeStruct((1, 1), jnp.float32),
        scratch_shapes=[pltpu.VMEM((d, b), jnp.bfloat16),
                        pltpu.VMEM((b, _BLK), jnp.bfloat16)],
    )(at, pt, pt)
    return out[0, 0]


# speedup vs baseline: 1.0747x; 1.0747x over previous
import jax
import jax.numpy as jnp
from jax import lax
from jax.experimental import pallas as pl
from jax.experimental.pallas import tpu as pltpu

_NUM_NEG = 4
_EPS = 1e-8
_BLK = 1024
_NEG_INF = -1e30


def _bce0(v):
    return jnp.maximum(v, 0.0) + jnp.log1p(jnp.exp(-jnp.abs(v)))


def _loss_kernel(a_ref, p_full_ref, p_blk_ref, out_ref, pnn_ref, work_ref):
    i = pl.program_id(0)
    nsteps = pl.num_programs(0)
    blk, b = _BLK, p_full_ref.shape[1]

    @pl.when(i == 0)
    def _prep():
        pt = p_full_ref[...]
        p_norm = jnp.sqrt(jnp.sum(pt * pt, axis=0, keepdims=True))
        p_nt = pt / p_norm
        pn_norm = jnp.sqrt(jnp.sum(p_nt * p_nt, axis=0, keepdims=True))
        pnn_ref[...] = (p_nt / pn_norm).astype(jnp.bfloat16)

    at = a_ref[...]
    a_norm = jnp.sqrt(jnp.sum(at * at, axis=0, keepdims=True))
    a_nt = (at / a_norm).astype(jnp.bfloat16)

    scaled = lax.dot_general(pnn_ref[...], a_nt, (((0,), (0,)), ((), ())),
                             preferred_element_type=jnp.float32
                             ).astype(jnp.bfloat16)

    work_ref[...] = scaled
    eye = (lax.broadcasted_iota(jnp.int32, (blk, blk), 0)
           == lax.broadcasted_iota(jnp.int32, (blk, blk), 1))
    sub = work_ref[pl.ds(i * blk, blk), :]
    work_ref[pl.ds(i * blk, blk), :] = jnp.where(
        eye, jnp.bfloat16(_NEG_INF), sub)
    work = work_ref[...]

    v1 = jnp.max(work, axis=0, keepdims=True)
    v2 = jnp.max(jnp.where(work < v1, work, jnp.bfloat16(_NEG_INF)),
                 axis=0, keepdims=True)
    v3 = jnp.max(jnp.where(work < v2, work, jnp.bfloat16(_NEG_INF)),
                 axis=0, keepdims=True)
    v4 = jnp.max(jnp.where(work < v3, work, jnp.bfloat16(_NEG_INF)),
                 axis=0, keepdims=True)
    v1, v2, v3, v4 = (v.astype(jnp.float32) for v in (v1, v2, v3, v4))

    pbt = p_blk_ref[...]
    num = jnp.sum(at * pbt, axis=0, keepdims=True)
    pb_norm = jnp.sqrt(jnp.sum(pbt * pbt, axis=0, keepdims=True))
    x = num / jnp.clip(a_norm * pb_norm, _EPS, None)
    pos_sum = jnp.sum(jnp.maximum(x, 0.0) - x
                      + jnp.log1p(jnp.exp(-jnp.abs(x))))

    t4 = jnp.where(v4 < 0.0, _bce0(x), _bce0(v4))
    neg_sum = jnp.sum(_bce0(v1) + _bce0(v2) + _bce0(v3) + t4)

    @pl.when(i == 0)
    def _init():
        out_ref[...] = jnp.zeros((1, 1), jnp.float32)

    out_ref[...] += jnp.full((1, 1), pos_sum + neg_sum, jnp.float32)

    @pl.when(i == nsteps - 1)
    def _finish():
        out_ref[...] = out_ref[...] / (b * (_NUM_NEG + 1))


def kernel(anchor, positive):
    b, d = positive.shape
    at = anchor.T
    pt = positive.T
    grid = b // _BLK
    out = pl.pallas_call(
        _loss_kernel,
        grid=(grid,),
        in_specs=[
            pl.BlockSpec((d, _BLK), lambda i: (0, i)),
            pl.BlockSpec((d, b), lambda i: (0, 0)),
            pl.BlockSpec((d, _BLK), lambda i: (0, i)),
        ],
        out_specs=pl.BlockSpec((1, 1), lambda i: (0, 0)),
        out_shape=jax.ShapeDtypeStruct((1, 1), jnp.float32),
        scratch_shapes=[pltpu.VMEM((d, b), jnp.bfloat16),
                        pltpu.VMEM((b, _BLK), jnp.bfloat16)],
    )(at, pt, pt)
    return out[0, 0]
